# baseline (speedup 1.0000x reference)
import functools

import jax
import jax.numpy as jnp
from jax import lax
from jax.experimental import pallas as pl
from jax.experimental.pallas import tpu as pltpu
from jax.experimental.pallas import tpu_sc as plsc

N_NODES = 10000
F = 128
E = 320000
EPAD = 331776
IDX_ROWS = EPAD // 128
ROWS_PER_TILE = 81
ACC_N = 10112
ZERO_ROWS = ACC_N // 16
OUT_ROWS = ACC_N // 16

NB = 3
RNDS = ROWS_PER_TILE // NB


@functools.cache
def _prop_kernel():
    mesh = plsc.VectorSubcoreMesh(core_axis_name="c", subcore_axis_name="s",
                                  num_cores=2, num_subcores=16)
    return pl.kernel(
        _prop_body,
        out_type=jax.ShapeDtypeStruct((2, ACC_N, F), jnp.float32),
        mesh=mesh,
        scratch_types=(
            [pltpu.VMEM((128,), jnp.int32)] * (2 * NB)
            + [pltpu.VMEM((128, F), jnp.float32)] * NB
            + [pltpu.VMEM_SHARED((ACC_N, F), jnp.float32)]
            + [pltpu.SemaphoreType.DMA] * (3 * NB)
        ),
    )


def _prop(h, srcp, dstp, zrows):
    return _prop_kernel()(h, srcp, dstp, zrows)


def _prop_body(h_hbm, srcr_hbm, dstr_hbm, zrows_hbm, out_hbm, *rest):
    srcw = rest[0:NB]
    dstw = rest[NB:2 * NB]
    rows = rest[2 * NB:3 * NB]
    acc = rest[3 * NB]
    isems = rest[3 * NB + 1:4 * NB + 1]
    gsems = rest[4 * NB + 1:5 * NB + 1]
    ssems = rest[5 * NB + 1:6 * NB + 1]
    c = lax.axis_index("c")
    s = lax.axis_index("s")
    base_e = (c * 16 + s) * ROWS_PER_TILE * 128
    pltpu.sync_copy(zrows_hbm, acc.at[pl.ds(s * ZERO_ROWS, ZERO_ROWS)])
    plsc.subcore_barrier()

    idummy = srcr_hbm.at[pl.ds(0, 128)]
    rdummy = h_hbm.at[pl.ds(0, 128)]

    def rnd(g, _):
        for b in range(NB):
            @pl.when(g > 0)
            def _():
                pltpu.make_async_copy(rdummy, rows[b], ssems[b]).wait()
            e = base_e + (g * NB + b) * 128
            pltpu.async_copy(srcr_hbm.at[pl.ds(e, 128)], srcw[b], isems[b])
            pltpu.async_copy(dstr_hbm.at[pl.ds(e, 128)], dstw[b], isems[b])
        for b in range(NB):
            pltpu.make_async_copy(idummy, srcw[b], isems[b]).wait()
            pltpu.make_async_copy(idummy, dstw[b], isems[b]).wait()
            pltpu.async_copy(h_hbm.at[srcw[b]], rows[b], gsems[b])
        for b in range(NB):
            pltpu.make_async_copy(rdummy, rows[b], gsems[b]).wait()
            pltpu.async_copy(rows[b], acc.at[dstw[b]], ssems[b], add=True)
        return ()

    lax.fori_loop(0, RNDS, rnd, (), unroll=False)
    for b in range(NB):
        pltpu.make_async_copy(rdummy, rows[b], ssems[b]).wait()
    plsc.subcore_barrier()
    pltpu.sync_copy(acc.at[pl.ds(s * OUT_ROWS, OUT_ROWS)],
                    out_hbm.at[c, pl.ds(s * OUT_ROWS, OUT_ROWS)])


@functools.cache
def _deg_kernel():
    mesh = plsc.VectorSubcoreMesh(core_axis_name="c", subcore_axis_name="s",
                                  num_cores=2, num_subcores=16)
    return pl.kernel(
        _deg_body,
        out_type=jax.ShapeDtypeStruct((2, ACC_N, F), jnp.float32),
        mesh=mesh,
        scratch_types=(
            [pltpu.VMEM((128,), jnp.int32)] * NB
            + [pltpu.VMEM((128, F), jnp.float32)]
            + [pltpu.VMEM_SHARED((ACC_N, F), jnp.float32)]
            + [pltpu.SemaphoreType.DMA] * (2 * NB)
        ),
    )


def _deg(onesrows, dstp, zrows):
    return _deg_kernel()(onesrows, dstp, zrows)


def _deg_body(ones_hbm, dstr_hbm, zrows_hbm, out_hbm, *rest):
    dstw = rest[0:NB]
    ones = rest[NB]
    acc = rest[NB + 1]
    isems = rest[NB + 2:2 * NB + 2]
    ssems = rest[2 * NB + 2:3 * NB + 2]
    c = lax.axis_index("c")
    s = lax.axis_index("s")
    base_e = (c * 16 + s) * ROWS_PER_TILE * 128
    pltpu.sync_copy(ones_hbm, ones)
    pltpu.sync_copy(zrows_hbm, acc.at[pl.ds(s * ZERO_ROWS, ZERO_ROWS)])
    plsc.subcore_barrier()

    idummy = dstr_hbm.at[pl.ds(0, 128)]

    def rnd(g, _):
        for b in range(NB):
            @pl.when(g > 0)
            def _():
                pltpu.make_async_copy(ones_hbm, ones, ssems[b]).wait()
            e = base_e + (g * NB + b) * 128
            pltpu.async_copy(dstr_hbm.at[pl.ds(e, 128)], dstw[b], isems[b])
        for b in range(NB):
            pltpu.make_async_copy(idummy, dstw[b], isems[b]).wait()
            pltpu.async_copy(ones, acc.at[dstw[b]], ssems[b], add=True)
        return ()

    lax.fori_loop(0, RNDS, rnd, (), unroll=False)
    for b in range(NB):
        pltpu.make_async_copy(ones_hbm, ones, ssems[b]).wait()
    plsc.subcore_barrier()
    pltpu.sync_copy(acc.at[pl.ds(s * OUT_ROWS, OUT_ROWS)],
                    out_hbm.at[c, pl.ds(s * OUT_ROWS, OUT_ROWS)])


def _dinv(dega_ref, degb_ref):
    return jax.lax.rsqrt(dega_ref[0, :, 0:1] + degb_ref[0, :, 0:1] + 1.0)


def _k1a(x_ref, w_ref, out_ref):
    out_ref[...] = jnp.dot(x_ref[...], w_ref[...],
                           preferred_element_type=jnp.float32)


def _k1b(h_ref, dega_ref, degb_ref, out_ref):
    out_ref[...] = h_ref[...] * _dinv(dega_ref, degb_ref)


def _k2(pa_ref, pb_ref, h_ref, dega_ref, degb_ref, b_ref, w_ref, out_ref):
    dinv = _dinv(dega_ref, degb_ref)
    g = jnp.maximum((pa_ref[0] + pb_ref[0] + h_ref[...]) * dinv
                    + b_ref[...], 0.0)
    out_ref[...] = jnp.dot(g, w_ref[...],
                           preferred_element_type=jnp.float32) * dinv


def _k3(pa_ref, pb_ref, h_ref, dega_ref, degb_ref, b_ref, w_ref, bfc_ref,
        out_ref):
    dinv = _dinv(dega_ref, degb_ref)
    g = jnp.maximum((pa_ref[0] + pb_ref[0] + h_ref[...]) * dinv
                    + b_ref[...], 0.0)
    out_ref[...] = jnp.dot(g, w_ref[...],
                           preferred_element_type=jnp.float32) + bfc_ref[...]


def _mm1a(x, W1):
    return pl.pallas_call(
        _k1a,
        grid=(10,),
        in_specs=[
            pl.BlockSpec((1000, F), lambda i: (i, 0)),
            pl.BlockSpec((F, F), lambda i: (0, 0)),
        ],
        out_specs=pl.BlockSpec((1000, F), lambda i: (i, 0)),
        out_shape=jax.ShapeDtypeStruct((N_NODES, F), jnp.float32),
    )(x, W1)


def _scale1(h, deg2):
    return pl.pallas_call(
        _k1b,
        grid=(10,),
        in_specs=[
            pl.BlockSpec((1000, F), lambda i: (i, 0)),
            pl.BlockSpec((1, 1000, F), lambda i: (0, i, 0)),
            pl.BlockSpec((1, 1000, F), lambda i: (1, i, 0)),
        ],
        out_specs=pl.BlockSpec((1000, F), lambda i: (i, 0)),
        out_shape=jax.ShapeDtypeStruct((N_NODES, F), jnp.float32),
    )(h, deg2, deg2)


def _mm2(p, h, deg2, b, W2):
    return pl.pallas_call(
        _k2,
        grid=(10,),
        in_specs=[
            pl.BlockSpec((1, 1000, F), lambda i: (0, i, 0)),
            pl.BlockSpec((1, 1000, F), lambda i: (1, i, 0)),
            pl.BlockSpec((1000, F), lambda i: (i, 0)),
            pl.BlockSpec((1, 1000, F), lambda i: (0, i, 0)),
            pl.BlockSpec((1, 1000, F), lambda i: (1, i, 0)),
            pl.BlockSpec((1, F), lambda i: (0, 0)),
            pl.BlockSpec((F, F), lambda i: (0, 0)),
        ],
        out_specs=pl.BlockSpec((1000, F), lambda i: (i, 0)),
        out_shape=jax.ShapeDtypeStruct((N_NODES, F), jnp.float32),
    )(p, p, h, deg2, deg2, b, W2)


def _mm3(p, h, deg2, b, Wfc, bfc):
    return pl.pallas_call(
        _k3,
        grid=(25,),
        in_specs=[
            pl.BlockSpec((1, 400, F), lambda i: (0, i, 0)),
            pl.BlockSpec((1, 400, F), lambda i: (1, i, 0)),
            pl.BlockSpec((400, F), lambda i: (i, 0)),
            pl.BlockSpec((1, 400, F), lambda i: (0, i, 0)),
            pl.BlockSpec((1, 400, F), lambda i: (1, i, 0)),
            pl.BlockSpec((1, F), lambda i: (0, 0)),
            pl.BlockSpec((F, N_NODES), lambda i: (0, 0)),
            pl.BlockSpec((1, N_NODES), lambda i: (0, 0)),
        ],
        out_specs=pl.BlockSpec((400, N_NODES), lambda i: (i, 0)),
        out_shape=jax.ShapeDtypeStruct((N_NODES, N_NODES), jnp.float32),
    )(p, p, h, deg2, deg2, b, Wfc, bfc)


def kernel(x, edge_index, W1, b1, W2, b2, Wfc, bfc):
    src = edge_index[0].astype(jnp.int32)
    dst = edge_index[1].astype(jnp.int32)
    pad = EPAD - E
    srcp = jnp.concatenate(
        [src, jnp.arange(pad, dtype=jnp.int32) % N_NODES])
    dstp = jnp.concatenate(
        [dst, N_NODES + (jnp.arange(pad, dtype=jnp.int32) % (ACC_N - N_NODES))])


    zrows = jnp.zeros((ZERO_ROWS, F), jnp.float32)
    onesrows = jnp.ones((128, F), jnp.float32)

    deg2 = _deg(onesrows, dstp, zrows)
    h1 = _mm1a(x, W1)
    h1p = _scale1(h1, deg2)
    p1 = _prop(h1p, srcp, dstp, zrows)
    h2p = _mm2(p1, h1p, deg2, b1.reshape(1, F), W2)
    p2 = _prop(h2p, srcp, dstp, zrows)
    out = _mm3(p2, h2p, deg2, b2.reshape(1, F), Wfc,
               bfc.reshape(1, N_NODES))
    return out


# speedup vs baseline: 1.0213x; 1.0213x over previous
import functools

import jax
import jax.numpy as jnp
from jax import lax
from jax.experimental import pallas as pl
from jax.experimental.pallas import tpu as pltpu
from jax.experimental.pallas import tpu_sc as plsc

N_NODES = 10000
F = 128
E = 320000
EPAD = 331776
CHUNK = 96
ROWS_PER_TILE = 108
ACC_N = 10112
ZERO_ROWS = ACC_N // 16
OUT_ROWS = ACC_N // 16

NB = 4
RNDS = ROWS_PER_TILE // NB


@functools.cache
def _prop_kernel():
    mesh = plsc.VectorSubcoreMesh(core_axis_name="c", subcore_axis_name="s",
                                  num_cores=2, num_subcores=16)
    return pl.kernel(
        _prop_body,
        out_type=jax.ShapeDtypeStruct((2, ACC_N, F), jnp.float32),
        mesh=mesh,
        scratch_types=(
            [pltpu.VMEM((CHUNK,), jnp.int32)] * (2 * NB)
            + [pltpu.VMEM((CHUNK, F), jnp.float32)] * NB
            + [pltpu.VMEM_SHARED((ACC_N, F), jnp.float32)]
            + [pltpu.SemaphoreType.DMA] * (3 * NB)
        ),
    )


def _prop(h, srcp, dstp, zrows):
    return _prop_kernel()(h, srcp, dstp, zrows)


def _prop_body(h_hbm, srcr_hbm, dstr_hbm, zrows_hbm, out_hbm, *rest):
    srcw = rest[0:NB]
    dstw = rest[NB:2 * NB]
    rows = rest[2 * NB:3 * NB]
    acc = rest[3 * NB]
    isems = rest[3 * NB + 1:4 * NB + 1]
    gsems = rest[4 * NB + 1:5 * NB + 1]
    ssems = rest[5 * NB + 1:6 * NB + 1]
    c = lax.axis_index("c")
    s = lax.axis_index("s")
    base_e = (c * 16 + s) * ROWS_PER_TILE * CHUNK
    pltpu.sync_copy(zrows_hbm, acc.at[pl.ds(s * ZERO_ROWS, ZERO_ROWS)])
    plsc.subcore_barrier()

    idummy = srcr_hbm.at[pl.ds(0, CHUNK)]
    rdummy = h_hbm.at[pl.ds(0, CHUNK)]

    def rnd(g, _):
        for b in range(NB):
            @pl.when(g > 0)
            def _():
                pltpu.make_async_copy(rdummy, rows[b], ssems[b]).wait()
            e = base_e + (g * NB + b) * CHUNK
            pltpu.async_copy(srcr_hbm.at[pl.ds(e, CHUNK)], srcw[b], isems[b])
            pltpu.async_copy(dstr_hbm.at[pl.ds(e, CHUNK)], dstw[b], isems[b])
        for b in range(NB):
            pltpu.make_async_copy(idummy, srcw[b], isems[b]).wait()
            pltpu.make_async_copy(idummy, dstw[b], isems[b]).wait()
            pltpu.async_copy(h_hbm.at[srcw[b]], rows[b], gsems[b])
        for b in range(NB):
            pltpu.make_async_copy(rdummy, rows[b], gsems[b]).wait()
            pltpu.async_copy(rows[b], acc.at[dstw[b]], ssems[b], add=True)
        return ()

    lax.fori_loop(0, RNDS, rnd, (), unroll=False)
    for b in range(NB):
        pltpu.make_async_copy(rdummy, rows[b], ssems[b]).wait()
    plsc.subcore_barrier()
    pltpu.sync_copy(acc.at[pl.ds(s * OUT_ROWS, OUT_ROWS)],
                    out_hbm.at[c, pl.ds(s * OUT_ROWS, OUT_ROWS)])


@functools.cache
def _deg_kernel():
    mesh = plsc.VectorSubcoreMesh(core_axis_name="c", subcore_axis_name="s",
                                  num_cores=2, num_subcores=16)
    return pl.kernel(
        _deg_body,
        out_type=jax.ShapeDtypeStruct((2, ACC_N, F), jnp.float32),
        mesh=mesh,
        scratch_types=(
            [pltpu.VMEM((CHUNK,), jnp.int32)] * NB
            + [pltpu.VMEM((CHUNK, F), jnp.float32)]
            + [pltpu.VMEM_SHARED((ACC_N, F), jnp.float32)]
            + [pltpu.SemaphoreType.DMA] * (2 * NB)
        ),
    )


def _deg(onesrows, dstp, zrows):
    return _deg_kernel()(onesrows, dstp, zrows)


def _deg_body(ones_hbm, dstr_hbm, zrows_hbm, out_hbm, *rest):
    dstw = rest[0:NB]
    ones = rest[NB]
    acc = rest[NB + 1]
    isems = rest[NB + 2:2 * NB + 2]
    ssems = rest[2 * NB + 2:3 * NB + 2]
    c = lax.axis_index("c")
    s = lax.axis_index("s")
    base_e = (c * 16 + s) * ROWS_PER_TILE * CHUNK
    pltpu.sync_copy(ones_hbm, ones)
    pltpu.sync_copy(zrows_hbm, acc.at[pl.ds(s * ZERO_ROWS, ZERO_ROWS)])
    plsc.subcore_barrier()

    idummy = dstr_hbm.at[pl.ds(0, CHUNK)]

    def rnd(g, _):
        for b in range(NB):
            @pl.when(g > 0)
            def _():
                pltpu.make_async_copy(ones_hbm, ones, ssems[b]).wait()
            e = base_e + (g * NB + b) * CHUNK
            pltpu.async_copy(dstr_hbm.at[pl.ds(e, CHUNK)], dstw[b], isems[b])
        for b in range(NB):
            pltpu.make_async_copy(idummy, dstw[b], isems[b]).wait()
            pltpu.async_copy(ones, acc.at[dstw[b]], ssems[b], add=True)
        return ()

    lax.fori_loop(0, RNDS, rnd, (), unroll=False)
    for b in range(NB):
        pltpu.make_async_copy(ones_hbm, ones, ssems[b]).wait()
    plsc.subcore_barrier()
    pltpu.sync_copy(acc.at[pl.ds(s * OUT_ROWS, OUT_ROWS)],
                    out_hbm.at[c, pl.ds(s * OUT_ROWS, OUT_ROWS)])


def _dinv(dega_ref, degb_ref):
    return jax.lax.rsqrt(dega_ref[0, :, 0:1] + degb_ref[0, :, 0:1] + 1.0)


def _k1a(x_ref, w_ref, out_ref):
    out_ref[...] = jnp.dot(x_ref[...], w_ref[...],
                           preferred_element_type=jnp.float32)


def _k1b(h_ref, dega_ref, degb_ref, out_ref):
    out_ref[...] = h_ref[...] * _dinv(dega_ref, degb_ref)


def _k2(pa_ref, pb_ref, h_ref, dega_ref, degb_ref, b_ref, w_ref, out_ref):
    dinv = _dinv(dega_ref, degb_ref)
    g = jnp.maximum((pa_ref[0] + pb_ref[0] + h_ref[...]) * dinv
                    + b_ref[...], 0.0)
    out_ref[...] = jnp.dot(g, w_ref[...],
                           preferred_element_type=jnp.float32) * dinv


def _k3(pa_ref, pb_ref, h_ref, dega_ref, degb_ref, b_ref, w_ref, bfc_ref,
        out_ref):
    dinv = _dinv(dega_ref, degb_ref)
    g = jnp.maximum((pa_ref[0] + pb_ref[0] + h_ref[...]) * dinv
                    + b_ref[...], 0.0)
    out_ref[...] = jnp.dot(g, w_ref[...],
                           preferred_element_type=jnp.float32) + bfc_ref[...]


def _mm1a(x, W1):
    return pl.pallas_call(
        _k1a,
        grid=(10,),
        in_specs=[
            pl.BlockSpec((1000, F), lambda i: (i, 0)),
            pl.BlockSpec((F, F), lambda i: (0, 0)),
        ],
        out_specs=pl.BlockSpec((1000, F), lambda i: (i, 0)),
        out_shape=jax.ShapeDtypeStruct((N_NODES, F), jnp.float32),
    )(x, W1)


def _scale1(h, deg2):
    return pl.pallas_call(
        _k1b,
        grid=(10,),
        in_specs=[
            pl.BlockSpec((1000, F), lambda i: (i, 0)),
            pl.BlockSpec((1, 1000, F), lambda i: (0, i, 0)),
            pl.BlockSpec((1, 1000, F), lambda i: (1, i, 0)),
        ],
        out_specs=pl.BlockSpec((1000, F), lambda i: (i, 0)),
        out_shape=jax.ShapeDtypeStruct((N_NODES, F), jnp.float32),
    )(h, deg2, deg2)


def _mm2(p, h, deg2, b, W2):
    return pl.pallas_call(
        _k2,
        grid=(10,),
        in_specs=[
            pl.BlockSpec((1, 1000, F), lambda i: (0, i, 0)),
            pl.BlockSpec((1, 1000, F), lambda i: (1, i, 0)),
            pl.BlockSpec((1000, F), lambda i: (i, 0)),
            pl.BlockSpec((1, 1000, F), lambda i: (0, i, 0)),
            pl.BlockSpec((1, 1000, F), lambda i: (1, i, 0)),
            pl.BlockSpec((1, F), lambda i: (0, 0)),
            pl.BlockSpec((F, F), lambda i: (0, 0)),
        ],
        out_specs=pl.BlockSpec((1000, F), lambda i: (i, 0)),
        out_shape=jax.ShapeDtypeStruct((N_NODES, F), jnp.float32),
    )(p, p, h, deg2, deg2, b, W2)


def _mm3(p, h, deg2, b, Wfc, bfc):
    return pl.pallas_call(
        _k3,
        grid=(25,),
        in_specs=[
            pl.BlockSpec((1, 400, F), lambda i: (0, i, 0)),
            pl.BlockSpec((1, 400, F), lambda i: (1, i, 0)),
            pl.BlockSpec((400, F), lambda i: (i, 0)),
            pl.BlockSpec((1, 400, F), lambda i: (0, i, 0)),
            pl.BlockSpec((1, 400, F), lambda i: (1, i, 0)),
            pl.BlockSpec((1, F), lambda i: (0, 0)),
            pl.BlockSpec((F, N_NODES), lambda i: (0, 0)),
            pl.BlockSpec((1, N_NODES), lambda i: (0, 0)),
        ],
        out_specs=pl.BlockSpec((400, N_NODES), lambda i: (i, 0)),
        out_shape=jax.ShapeDtypeStruct((N_NODES, N_NODES), jnp.float32),
    )(p, p, h, deg2, deg2, b, Wfc, bfc)


def kernel(x, edge_index, W1, b1, W2, b2, Wfc, bfc):
    src = edge_index[0].astype(jnp.int32)
    dst = edge_index[1].astype(jnp.int32)
    pad = EPAD - E
    srcp = jnp.concatenate(
        [src, jnp.arange(pad, dtype=jnp.int32) % N_NODES])
    dstp = jnp.concatenate(
        [dst, N_NODES + (jnp.arange(pad, dtype=jnp.int32) % (ACC_N - N_NODES))])


    zrows = jnp.zeros((ZERO_ROWS, F), jnp.float32)
    onesrows = jnp.ones((CHUNK, F), jnp.float32)

    deg2 = _deg(onesrows, dstp, zrows)
    h1p = _scale1(_mm1a(x, W1), deg2)
    p1 = _prop(h1p, srcp, dstp, zrows)
    h2p = _mm2(p1, h1p, deg2, b1.reshape(1, F), W2)
    p2 = _prop(h2p, srcp, dstp, zrows)
    out = _mm3(p2, h2p, deg2, b2.reshape(1, F), Wfc,
               bfc.reshape(1, N_NODES))
    return out
